# Initial kernel scaffold; baseline (speedup 1.0000x reference)
import jax
import jax.numpy as jnp
from jax.experimental import pallas as pl


def kernel(x, edge_index, W1, b1, W2, b2):
    raise NotImplementedError("write your pallas kernel here")


# speedup vs baseline: 8.1483x; 8.1483x over previous
import functools
import math

import jax
import jax.numpy as jnp
from jax import lax
from jax.experimental import pallas as pl
from jax.experimental.pallas import tpu as pltpu
from jax.experimental.pallas import tpu_sc as plsc

_NUM_CORES = 2
_NUM_TILES = 16
_CHUNK = 128


@functools.lru_cache(maxsize=None)
def _make_deg_kernel(C, n_deg):
  rpt = n_deg // _NUM_TILES
  mesh = plsc.VectorSubcoreMesh(core_axis_name="c", subcore_axis_name="s")

  @functools.partial(
      pl.kernel,
      out_type=jax.ShapeDtypeStruct((_NUM_CORES, n_deg, _CHUNK), jnp.float32),
      mesh=mesh,
      scratch_types=[
          pltpu.VMEM((C, _CHUNK), jnp.int32),
          pltpu.VMEM((_CHUNK, _CHUNK), jnp.float32),
          pltpu.VMEM_SHARED((n_deg, _CHUNK), jnp.float32),
      ],
  )
  def deg_kernel(dst_hbm, out_hbm, dst_v, ones_v, acc):
    c = lax.axis_index("c")
    s = lax.axis_index("s")
    slab = c * _NUM_TILES + s
    pltpu.sync_copy(dst_hbm.at[slab], dst_v)

    @pl.loop(0, _CHUNK)
    def _(i):
      @pl.loop(0, _CHUNK // 16)
      def _(q):
        ones_v[i, pl.ds(q * 16, 16)] = jnp.zeros((16,), jnp.float32)

    r0 = s * rpt

    @pl.loop(0, rpt // _CHUNK)
    def _(k):
      pltpu.sync_copy(ones_v, acc.at[pl.ds(r0 + k * _CHUNK, _CHUNK)])

    rem = rpt % _CHUNK
    if rem:
      pltpu.sync_copy(ones_v.at[pl.ds(0, rem)],
                      acc.at[pl.ds(r0 + rpt - rem, rem)])

    @pl.loop(0, _CHUNK)
    def _(i):
      @pl.loop(0, _CHUNK // 16)
      def _(q):
        ones_v[i, pl.ds(q * 16, 16)] = jnp.ones((16,), jnp.float32)

    plsc.subcore_barrier()

    @pl.loop(0, C)
    def _(j):
      pltpu.sync_copy(ones_v, acc.at[dst_v.at[j]], add=True)

    plsc.subcore_barrier()
    pltpu.sync_copy(acc.at[pl.ds(r0, rpt)], out_hbm.at[c, pl.ds(r0, rpt)])

  return deg_kernel


@functools.lru_cache(maxsize=None)
def _make_agg_kernel(C, half, d):
  acc_rows = half + _CHUNK
  rpt = half // _NUM_TILES
  mesh = plsc.VectorSubcoreMesh(core_axis_name="c", subcore_axis_name="s")

  @functools.partial(
      pl.kernel,
      out_type=jax.ShapeDtypeStruct((_NUM_CORES, half, d), jnp.float32),
      mesh=mesh,
      scratch_types=[
          pltpu.VMEM((C, _CHUNK), jnp.int32),
          pltpu.VMEM((C, _CHUNK), jnp.int32),
          pltpu.VMEM((_CHUNK, d), jnp.float32),
          pltpu.VMEM((_CHUNK, d), jnp.float32),
          pltpu.VMEM((_CHUNK, d), jnp.float32),
          pltpu.VMEM_SHARED((acc_rows, d), jnp.float32),
          pltpu.SemaphoreType.DMA,
          pltpu.SemaphoreType.DMA,
      ],
  )
  def agg_kernel(u_hbm, src_hbm, dst_hbm, out_hbm,
                 src_v, dst_v, rows0, rows1, zero_v, acc, sem0, sem1):
    c = lax.axis_index("c")
    s = lax.axis_index("s")
    pltpu.sync_copy(src_hbm.at[s], src_v)
    pltpu.sync_copy(dst_hbm.at[c, s], dst_v)

    @pl.loop(0, _CHUNK)
    def _(i):
      @pl.loop(0, d // 16)
      def _(q):
        zero_v[i, pl.ds(q * 16, 16)] = jnp.zeros((16,), jnp.float32)

    r0 = s * rpt

    @pl.loop(0, rpt // _CHUNK)
    def _(k):
      pltpu.sync_copy(zero_v, acc.at[pl.ds(r0 + k * _CHUNK, _CHUNK)])

    rem = rpt % _CHUNK
    if rem:
      pltpu.sync_copy(zero_v.at[pl.ds(0, rem)],
                      acc.at[pl.ds(r0 + rpt - rem, rem)])
    plsc.subcore_barrier()

    @pl.loop(0, C)
    def _(j):
      pltpu.async_copy(u_hbm.at[src_v.at[j]], rows0, sem0).wait()
      pltpu.sync_copy(rows0, acc.at[dst_v.at[j]], add=True)

    plsc.subcore_barrier()
    pltpu.sync_copy(acc.at[pl.ds(r0, rpt)], out_hbm.at[c, pl.ds(r0, rpt)])

  return agg_kernel


def _dinv(deg_ref):
  return lax.rsqrt(deg_ref[0, :, 0:1] + deg_ref[1, :, 0:1] + 1.0)


def _tc_a_body(x_ref, w_ref, deg_ref, u_ref):
  h = jnp.dot(x_ref[...], w_ref[...], preferred_element_type=jnp.float32)
  u_ref[...] = h * _dinv(deg_ref)


def _tc_b_body(p_ref, u_ref, deg_ref, b_ref, o_ref):
  dinv = _dinv(deg_ref)
  agg = (p_ref[...] + u_ref[...]) * dinv + b_ref[...]
  o_ref[...] = jnp.maximum(agg, 0.0) * dinv


def _tc_c_body(q_ref, u_ref, deg_ref, b_ref, w_ref, z_ref):
  g = (q_ref[...] + u_ref[...]) * _dinv(deg_ref)
  z_ref[...] = (
      jnp.dot(g, w_ref[...], preferred_element_type=jnp.float32) + b_ref[...])


def _row_block(n):
  for r in (1024, 1000, 512, 500, 256, 250, 128, 8):
    if n % r == 0:
      return r
  return n


def _tc_a(x, w1, dinv):
  n, d_in = x.shape
  d_h = w1.shape[1]
  r = _row_block(n)
  return pl.pallas_call(
      _tc_a_body,
      grid=(n // r,),
      in_specs=[
          pl.BlockSpec((r, d_in), lambda i: (i, 0)),
          pl.BlockSpec((d_in, d_h), lambda i: (0, 0)),
          pl.BlockSpec((2, r, _CHUNK), lambda i: (0, i, 0)),
      ],
      out_specs=pl.BlockSpec((r, d_h), lambda i: (i, 0)),
      out_shape=jax.ShapeDtypeStruct((n, d_h), jnp.float32),
  )(x, w1, dinv)


def _tc_b(p, u1, dinv, b1):
  n, d_h = u1.shape
  r = _row_block(n)
  return pl.pallas_call(
      _tc_b_body,
      grid=(n // r,),
      in_specs=[
          pl.BlockSpec((r, d_h), lambda i: (i, 0)),
          pl.BlockSpec((r, d_h), lambda i: (i, 0)),
          pl.BlockSpec((2, r, _CHUNK), lambda i: (0, i, 0)),
          pl.BlockSpec((1, d_h), lambda i: (0, 0)),
      ],
      out_specs=pl.BlockSpec((r, d_h), lambda i: (i, 0)),
      out_shape=jax.ShapeDtypeStruct((n, d_h), jnp.float32),
  )(p, u1, dinv, b1)


def _tc_c(q, v, dinv, b2, w2):
  n, d_h = v.shape
  d_o = w2.shape[1]
  r = _row_block(n)
  return pl.pallas_call(
      _tc_c_body,
      grid=(n // r,),
      in_specs=[
          pl.BlockSpec((r, d_h), lambda i: (i, 0)),
          pl.BlockSpec((r, d_h), lambda i: (i, 0)),
          pl.BlockSpec((2, r, _CHUNK), lambda i: (0, i, 0)),
          pl.BlockSpec((1, d_o), lambda i: (0, 0)),
          pl.BlockSpec((d_h, d_o), lambda i: (0, 0)),
      ],
      out_specs=pl.BlockSpec((r, d_o), lambda i: (i, 0)),
      out_shape=jax.ShapeDtypeStruct((n, d_o), jnp.float32),
  )(q, v, dinv, b2, w2)


def kernel(x, edge_index, W1, b1, W2, b2):
  n = x.shape[0]
  e = edge_index.shape[1]

  half = (_NUM_TILES * 32) * math.ceil((n + 2) / 2 / (_NUM_TILES * 32))
  n_pad = 2 * half
  trash = half

  chunks = math.ceil(e / (_NUM_TILES * _CHUNK))
  chunks += chunks % 2
  e_pad = _NUM_TILES * chunks * _CHUNK
  pad = e_pad - e

  src = edge_index[0]
  dst = edge_index[1]
  src_p = jnp.concatenate(
      [src, jnp.zeros((pad,), jnp.int32)]).reshape(_NUM_TILES, chunks, _CHUNK)
  dst_f = jnp.concatenate([dst, jnp.full((pad,), n, jnp.int32)])
  dst_lo = jnp.where(dst_f < half, dst_f, trash)
  dst_hi_raw = dst_f - half
  dst_hi = jnp.where(dst_f >= half, dst_hi_raw, trash)
  dst_p = jnp.stack([dst_lo, dst_hi]).reshape(
      _NUM_CORES, _NUM_TILES, chunks, _CHUNK)

  n_deg = _CHUNK * math.ceil((n + 1) / _CHUNK)
  dst_deg = dst_f.reshape(_NUM_CORES * _NUM_TILES, chunks // 2, _CHUNK)
  dinv = _make_deg_kernel(chunks // 2, n_deg)(dst_deg)

  u1 = _tc_a(x, W1, dinv)
  agg = _make_agg_kernel(chunks, half, u1.shape[1])
  p = agg(u1, src_p, dst_p)
  v = _tc_b(p.reshape(n_pad, -1), u1, dinv, b1.reshape(1, -1))
  q = agg(v, src_p, dst_p)
  z = _tc_c(q.reshape(n_pad, -1), v, dinv, b2.reshape(1, -1), W2)
  return z
